# Initial kernel scaffold; baseline (speedup 1.0000x reference)
import jax
import jax.numpy as jnp
from jax.experimental import pallas as pl


def kernel(word_id, tag_id, pos_1, pos_2, word_table, tag_table, pos_table):
    raise NotImplementedError("write your pallas kernel here")


# speedup vs baseline: 4.4948x; 4.4948x over previous
import functools

import jax
import jax.numpy as jnp
from jax import lax
from jax.experimental import pallas as pl
from jax.experimental.pallas import tpu as pltpu
from jax.experimental.pallas import tpu_sc as plsc

WD, TD, PD = 128, 32, 32
OUT_D = WD + TD + PD + PD
NC, NS = 2, 16
NW = NC * NS


@functools.partial(jax.jit, static_argnames=("T", "C", "nchunk"))
def _emb_call(idx, word_table, tag_table, pos_table, T, C, nchunk):
    tpw = T // NW
    mesh = plsc.VectorSubcoreMesh(core_axis_name="c", subcore_axis_name="s")

    @functools.partial(
        pl.kernel,
        out_type=jax.ShapeDtypeStruct((T, OUT_D), jnp.float32),
        mesh=mesh,
        scratch_types=[
            pltpu.VMEM((4, nchunk, C), jnp.int32),
            pltpu.VMEM((C, WD), jnp.float32),
            pltpu.VMEM((C, TD), jnp.float32),
            pltpu.VMEM((C, PD), jnp.float32),
            pltpu.VMEM((C, PD), jnp.float32),
            pltpu.SemaphoreType.DMA,
        ],
        compiler_params=pltpu.CompilerParams(use_tc_tiling_on_sc=False),
    )
    def emb(idx_hbm, wt_hbm, tt_hbm, pt_hbm, out_hbm,
            idx_v, wbuf, tbuf, p1buf, p2buf, sem):
        wid = lax.axis_index("s") * NC + lax.axis_index("c")
        pltpu.sync_copy(idx_hbm.at[wid], idx_v)

        @pl.loop(0, nchunk)
        def chunk_loop(i):
            base = wid * tpw + i * C
            cw = pltpu.async_copy(wt_hbm.at[idx_v.at[0, i]], wbuf, sem)
            ct = pltpu.async_copy(tt_hbm.at[idx_v.at[1, i]], tbuf, sem)
            c1 = pltpu.async_copy(pt_hbm.at[idx_v.at[2, i]], p1buf, sem)
            c2 = pltpu.async_copy(pt_hbm.at[idx_v.at[3, i]], p2buf, sem)
            cw.wait()
            ct.wait()
            c1.wait()
            c2.wait()
            pltpu.sync_copy(wbuf, out_hbm.at[pl.ds(base, C), pl.ds(0, WD)])
            pltpu.sync_copy(tbuf, out_hbm.at[pl.ds(base, C), pl.ds(WD, TD)])
            pltpu.sync_copy(p1buf, out_hbm.at[pl.ds(base, C), pl.ds(WD + TD, PD)])
            pltpu.sync_copy(p2buf, out_hbm.at[pl.ds(base, C), pl.ds(WD + TD + PD, PD)])

    return emb(idx, word_table, tag_table, pos_table)


def kernel(word_id, tag_id, pos_1, pos_2, word_table, tag_table, pos_table):
    B, L = word_id.shape
    T = B * L
    C = 128
    nchunk = T // (NW * C)
    idx = jnp.stack([
        word_id.reshape(T).astype(jnp.int32),
        tag_id.reshape(T).astype(jnp.int32),
        pos_1.reshape(T).astype(jnp.int32),
        pos_2.reshape(T).astype(jnp.int32),
    ])
    idx = idx.reshape(4, NW, nchunk, C).transpose(1, 0, 2, 3)
    out = _emb_call(idx, word_table, tag_table, pos_table, T, C, nchunk)
    return out.reshape(B, L, OUT_D)
